# baseline (speedup 1.0000x reference)
import functools

import jax
import jax.numpy as jnp
from jax import lax
from jax.experimental import pallas as pl
from jax.experimental.pallas import tpu as pltpu
from jax.experimental.pallas import tpu_sc as plsc

D = 64
GROUP = 128
NC, NS = 2, 16
NW = NC * NS
NBUF = 8


@functools.lru_cache(maxsize=None)
def _build(B):
    assert B % (GROUP * NW) == 0
    n_groups = B // GROUP
    g_per_w = n_groups // NW
    assert g_per_w % NBUF == 0
    k_iters = g_per_w // NBUF

    mesh = plsc.VectorSubcoreMesh(core_axis_name="c", subcore_axis_name="s")

    @functools.partial(
        pl.kernel,
        out_type=jax.ShapeDtypeStruct((B, D), jnp.float32),
        mesh=mesh,
        compiler_params=pltpu.CompilerParams(use_tc_tiling_on_sc=False),
        scratch_types=[
            pltpu.VMEM((g_per_w, GROUP), jnp.int32),
            pltpu.VMEM((NBUF, GROUP, D), jnp.float32),
        ]
        + [pltpu.SemaphoreType.DMA] * (2 * NBUF),
    )
    def emb_kernel(idx_hbm, table_hbm, out_hbm, idx_v, rows_v, *sems):
        gsems = sems[:NBUF]
        wsems = sems[NBUF:]
        wid = lax.axis_index("s") * NC + lax.axis_index("c")
        g0 = wid * g_per_w

        pltpu.sync_copy(idx_hbm.at[pl.ds(g0, g_per_w)], idx_v)

        def gather(j, b):
            return pltpu.make_async_copy(
                table_hbm.at[idx_v.at[j]], rows_v.at[b], gsems[b]
            )

        def write(j, b):
            return pltpu.make_async_copy(
                rows_v.at[b], out_hbm.at[pl.ds((g0 + j) * GROUP, GROUP)], wsems[b]
            )

        for b in range(NBUF):
            gather(b, b).start()

        def body(k, _):
            base = k * NBUF
            for b in range(NBUF):
                j = base + b
                gather(j, b).wait()
                write(j, b).start()
                if b >= 2:
                    b2 = b - 2
                    j2 = base + b2
                    write(j2, b2).wait()
                    gather(j2 + NBUF, b2).start()
            for b2 in (NBUF - 2, NBUF - 1):
                j2 = base + b2
                write(j2, b2).wait()
                gather(j2 + NBUF, b2).start()
            return 0

        lax.fori_loop(0, k_iters - 1, body, 0)

        base = (k_iters - 1) * NBUF
        for b in range(NBUF):
            j = base + b
            gather(j, b).wait()
            write(j, b).start()
        for b in range(NBUF):
            write(base + b, b).wait()

    return emb_kernel


def kernel(token_ids, emb):
    s0, s1 = token_ids.shape
    B = s0 * s1
    idx = token_ids.reshape(B // GROUP, GROUP).astype(jnp.int32)
    out = _build(B)(idx, emb)
    return out.reshape(s0, s1, D)
